# baseline (speedup 1.0000x reference)
import functools

import jax
import jax.numpy as jnp
from jax import lax
from jax.experimental import pallas as pl
from jax.experimental.pallas import tpu as pltpu
from jax.experimental.pallas import tpu_sc as plsc

FEAT = 16
TEX = 512
L = 16
NW = 32
CHUNK = 128
W_OUT = 512
QUARTERS = W_OUT // CHUNK
OPITCH = W_OUT + 1


def _bcast(vec, p):
    idx = jnp.full((L, 1), p, jnp.int32)
    return lax.gather(
        vec, idx,
        lax.GatherDimensionNumbers(
            offset_dims=(), collapsed_slice_dims=(0,), start_index_map=(0,)),
        (1,), mode=lax.GatherScatterMode.PROMISE_IN_BOUNDS)


def _body(xs_hbm, ys_hbm, tab_hbm, out_hbm,
          xs_v, ys_v, i00, i01, i10, i11, w00, w01, w10, w11,
          r00, r01, r10, r11, obuf, sems, *, rows_per_w):
    cid = lax.axis_index("c")
    sid = lax.axis_index("s")
    wid = sid * 2 + cid

    def row_loop(r, _):
        row = wid * rows_per_w + r
        n = row // TEX
        h = row % TEX
        base = row * W_OUT

        pltpu.sync_copy(xs_hbm.at[pl.ds(base, W_OUT)], xs_v)
        pltpu.sync_copy(ys_hbm.at[pl.ds(base, W_OUT)], ys_v)

        for q in range(QUARTERS):
            @plsc.parallel_loop(0, CHUNK // L, 1, unroll=2)
            def stage1(g):
                gsl = pl.ds(q * CHUNK + g * L, L)
                xv = xs_v[gsl]
                yv = ys_v[gsl]
                gx = xv * 2.0 - 1.0
                gy = yv * 2.0 - 1.0
                ix = ((gx + 1.0) * TEX - 1.0) * 0.5
                iy = ((gy + 1.0) * TEX - 1.0) * 0.5
                ix0 = (ix + 1.0).astype(jnp.int32) - 1
                iy0 = (iy + 1.0).astype(jnp.int32) - 1
                fx = ix - ix0.astype(jnp.float32)
                fy = iy - iy0.astype(jnp.float32)
                ix1 = ix0 + 1
                iy1 = iy0 + 1
                zero = jnp.zeros((L,), jnp.float32)
                wx0 = jnp.where(ix0 >= 0, 1.0 - fx, zero)
                wx1 = jnp.where(ix1 <= TEX - 1, fx, zero)
                wy0 = jnp.where(iy0 >= 0, 1.0 - fy, zero)
                wy1 = jnp.where(iy1 <= TEX - 1, fy, zero)
                cx0 = jnp.maximum(ix0, 0)
                cx1 = jnp.minimum(ix1, TEX - 1)
                ry0 = jnp.maximum(iy0, 0) * TEX
                ry1 = jnp.minimum(iy1, TEX - 1) * TEX
                sl = pl.ds(g * L, L)
                i00[q, sl] = ry0 + cx0
                i01[q, sl] = ry0 + cx1
                i10[q, sl] = ry1 + cx0
                i11[q, sl] = ry1 + cx1
                w00[gsl] = wy0 * wx0
                w01[gsl] = wy0 * wx1
                w10[gsl] = wy1 * wx0
                w11[gsl] = wy1 * wx1

            rsl = pl.ds(q * CHUNK, CHUNK)
            pltpu.async_copy(tab_hbm.at[i00.at[q]], r00.at[rsl], sems.at[q])
            pltpu.async_copy(tab_hbm.at[i01.at[q]], r01.at[rsl], sems.at[q])
            pltpu.async_copy(tab_hbm.at[i10.at[q]], r10.at[rsl], sems.at[q])
            pltpu.async_copy(tab_hbm.at[i11.at[q]], r11.at[rsl], sems.at[q])

        for q in range(QUARTERS):
            rsl = pl.ds(q * CHUNK, CHUNK)
            for rbuf, ibuf in ((r00, i00), (r01, i01), (r10, i10), (r11, i11)):
                pltpu.make_async_copy(
                    tab_hbm.at[ibuf.at[q]], rbuf.at[rsl], sems.at[q]).wait()

            @plsc.parallel_loop(0, CHUNK // L, 1, unroll=2)
            def stage2(g):
                sl = pl.ds(q * CHUNK + g * L, L)
                a00 = w00[sl]
                a01 = w01[sl]
                a10 = w10[sl]
                a11 = w11[sl]
                lanes = lax.iota(jnp.int32, L)
                col0 = jnp.full((L,), q * CHUNK + g * L, jnp.int32)
                for p in range(L):
                    b00 = _bcast(a00, p)
                    b01 = _bcast(a01, p)
                    b10 = _bcast(a10, p)
                    b11 = _bcast(a11, p)
                    pt = q * CHUNK + g * L + p
                    v00 = r00[pt]
                    v01 = r01[pt]
                    v10 = r10[pt]
                    v11 = r11[pt]
                    acc = b00 * v00 + b01 * v01 + b10 * v10 + b11 * v11
                    plsc.store_scatter(obuf, [lanes, col0 + p], acc)

        pltpu.sync_copy(obuf.at[:, pl.ds(0, W_OUT)], out_hbm.at[n, :, h, :])
        return 0

    lax.fori_loop(0, rows_per_w, row_loop, 0)


def kernel(x, textures):
    batch = x.shape[0]
    rows = batch * TEX
    rows_per_w = rows // NW

    xs = x[..., 0].reshape(-1)
    ys = x[..., 1].reshape(-1)
    tab = textures.reshape(FEAT, TEX * TEX).T

    mesh = plsc.VectorSubcoreMesh(core_axis_name="c", subcore_axis_name="s")
    f = pl.kernel(
        functools.partial(_body, rows_per_w=rows_per_w),
        out_type=jax.ShapeDtypeStruct((batch, FEAT, TEX, TEX), jnp.float32),
        mesh=mesh,
        compiler_params=pltpu.CompilerParams(
            needs_layout_passes=False, use_tc_tiling_on_sc=False),
        scratch_types=[
            pltpu.VMEM((W_OUT,), jnp.float32),
            pltpu.VMEM((W_OUT,), jnp.float32),
            pltpu.VMEM((QUARTERS, CHUNK), jnp.int32),
            pltpu.VMEM((QUARTERS, CHUNK), jnp.int32),
            pltpu.VMEM((QUARTERS, CHUNK), jnp.int32),
            pltpu.VMEM((QUARTERS, CHUNK), jnp.int32),
            pltpu.VMEM((W_OUT,), jnp.float32),
            pltpu.VMEM((W_OUT,), jnp.float32),
            pltpu.VMEM((W_OUT,), jnp.float32),
            pltpu.VMEM((W_OUT,), jnp.float32),
            pltpu.VMEM((W_OUT, FEAT), jnp.float32),
            pltpu.VMEM((W_OUT, FEAT), jnp.float32),
            pltpu.VMEM((W_OUT, FEAT), jnp.float32),
            pltpu.VMEM((W_OUT, FEAT), jnp.float32),
            pltpu.VMEM((FEAT, OPITCH), jnp.float32),
            pltpu.SemaphoreType.DMA((QUARTERS,)),
        ],
    )
    return f(xs, ys, tab)


# speedup vs baseline: 1.4073x; 1.4073x over previous
import functools

import jax
import jax.numpy as jnp
from jax import lax
from jax.experimental import pallas as pl
from jax.experimental.pallas import tpu as pltpu
from jax.experimental.pallas import tpu_sc as plsc

FEAT = 16
TEX = 512
L = 16
NW = 32
CHUNK = 128
W_OUT = 512
QUARTERS = W_OUT // CHUNK
OPITCH = W_OUT + 1


def _bcast(vec, p):
    idx = jnp.full((L, 1), p, jnp.int32)
    return lax.gather(
        vec, idx,
        lax.GatherDimensionNumbers(
            offset_dims=(), collapsed_slice_dims=(0,), start_index_map=(0,)),
        (1,), mode=lax.GatherScatterMode.PROMISE_IN_BOUNDS)


def _body(xs_hbm, ys_hbm, tab_hbm, out_hbm, *scr, rows_per_w):
    bufs = [scr[0:15], scr[15:30]]
    gsem, csem, osem = scr[30], scr[31], scr[32]
    cid = lax.axis_index("c")
    sid = lax.axis_index("s")
    wid = sid * 2 + cid
    row0 = wid * rows_per_w

    def coords_fire(rglob, par):
        xsv, ysv = bufs[par][0], bufs[par][1]
        base = rglob * W_OUT
        pltpu.async_copy(xs_hbm.at[pl.ds(base, W_OUT)], xsv, csem.at[par])
        pltpu.async_copy(ys_hbm.at[pl.ds(base, W_OUT)], ysv, csem.at[par])

    def coords_wait(rglob, par):
        xsv, ysv = bufs[par][0], bufs[par][1]
        base = rglob * W_OUT
        pltpu.make_async_copy(
            xs_hbm.at[pl.ds(base, W_OUT)], xsv, csem.at[par]).wait()
        pltpu.make_async_copy(
            ys_hbm.at[pl.ds(base, W_OUT)], ysv, csem.at[par]).wait()

    def do_row(r_local, par):
        xsv, ysv = bufs[par][0], bufs[par][1]
        ibs = bufs[par][2:6]
        wbs = bufs[par][6:10]
        rbs = bufs[par][10:14]
        ob = bufs[par][14]
        i00, i01, i10, i11 = ibs
        w00, w01, w10, w11 = wbs
        r00, r01, r10, r11 = rbs

        rglob = row0 + r_local
        n = rglob // TEX
        h = rglob % TEX

        coords_wait(rglob, par)

        @pl.when(r_local + 1 < rows_per_w)
        def _():
            coords_fire(row0 + r_local + 1, 1 - par)

        for q in range(QUARTERS):
            def stage1(g, _):
                gsl = pl.ds(q * CHUNK + g * L, L)
                xv = xsv[gsl]
                yv = ysv[gsl]
                gx = xv * 2.0 - 1.0
                gy = yv * 2.0 - 1.0
                ix = ((gx + 1.0) * TEX - 1.0) * 0.5
                iy = ((gy + 1.0) * TEX - 1.0) * 0.5
                ix0 = (ix + 1.0).astype(jnp.int32) - 1
                iy0 = (iy + 1.0).astype(jnp.int32) - 1
                fx = ix - ix0.astype(jnp.float32)
                fy = iy - iy0.astype(jnp.float32)
                ix1 = ix0 + 1
                iy1 = iy0 + 1
                zero = jnp.zeros((L,), jnp.float32)
                wx0 = jnp.where(ix0 >= 0, 1.0 - fx, zero)
                wx1 = jnp.where(ix1 <= TEX - 1, fx, zero)
                wy0 = jnp.where(iy0 >= 0, 1.0 - fy, zero)
                wy1 = jnp.where(iy1 <= TEX - 1, fy, zero)
                cx0 = jnp.maximum(ix0, 0)
                cx1 = jnp.minimum(ix1, TEX - 1)
                ry0 = jnp.maximum(iy0, 0) * TEX
                ry1 = jnp.minimum(iy1, TEX - 1) * TEX
                sl = pl.ds(g * L, L)
                i00[q, sl] = ry0 + cx0
                i01[q, sl] = ry0 + cx1
                i10[q, sl] = ry1 + cx0
                i11[q, sl] = ry1 + cx1
                w00[gsl] = wy0 * wx0
                w01[gsl] = wy0 * wx1
                w10[gsl] = wy1 * wx0
                w11[gsl] = wy1 * wx1
                return 0

            lax.fori_loop(0, CHUNK // L, stage1, 0)

            rsl = pl.ds(q * CHUNK, CHUNK)
            pltpu.async_copy(tab_hbm.at[i00.at[q]], r00.at[rsl], gsem.at[par, q])
            pltpu.async_copy(tab_hbm.at[i01.at[q]], r01.at[rsl], gsem.at[par, q])
            pltpu.async_copy(tab_hbm.at[i10.at[q]], r10.at[rsl], gsem.at[par, q])
            pltpu.async_copy(tab_hbm.at[i11.at[q]], r11.at[rsl], gsem.at[par, q])

        @pl.when(r_local >= 2)
        def _():
            pltpu.make_async_copy(
                ob.at[:, pl.ds(0, W_OUT)], out_hbm.at[n, :, h, :],
                osem.at[par]).wait()

        for q in range(QUARTERS):
            rsl = pl.ds(q * CHUNK, CHUNK)
            for rbuf, ibuf in ((r00, i00), (r01, i01), (r10, i10), (r11, i11)):
                pltpu.make_async_copy(
                    tab_hbm.at[ibuf.at[q]], rbuf.at[rsl],
                    gsem.at[par, q]).wait()

            def stage2(g, _):
                sl = pl.ds(q * CHUNK + g * L, L)
                a00 = w00[sl]
                a01 = w01[sl]
                a10 = w10[sl]
                a11 = w11[sl]
                lanes = lax.iota(jnp.int32, L)
                col0 = jnp.full((L,), q * CHUNK + g * L, jnp.int32)
                for p in range(L):
                    b00 = _bcast(a00, p)
                    b01 = _bcast(a01, p)
                    b10 = _bcast(a10, p)
                    b11 = _bcast(a11, p)
                    pt = q * CHUNK + g * L + p
                    v00 = r00[pt]
                    v01 = r01[pt]
                    v10 = r10[pt]
                    v11 = r11[pt]
                    acc = b00 * v00 + b01 * v01 + b10 * v10 + b11 * v11
                    plsc.store_scatter(ob, [lanes, col0 + p], acc)
                return 0

            lax.fori_loop(0, CHUNK // L, stage2, 0)

        pltpu.async_copy(ob.at[:, pl.ds(0, W_OUT)], out_hbm.at[n, :, h, :],
                         osem.at[par])

    coords_fire(row0, 0)

    def row_pair(i, _):
        do_row(2 * i, 0)
        do_row(2 * i + 1, 1)
        return 0

    lax.fori_loop(0, rows_per_w // 2, row_pair, 0)

    for par in (0, 1):
        r_local = rows_per_w - 2 + par
        rglob = row0 + r_local
        n = rglob // TEX
        h = rglob % TEX
        ob = bufs[par][14]
        pltpu.make_async_copy(
            ob.at[:, pl.ds(0, W_OUT)], out_hbm.at[n, :, h, :],
            osem.at[par]).wait()


def kernel(x, textures):
    batch = x.shape[0]
    rows = batch * TEX
    rows_per_w = rows // NW

    xs = x[..., 0].reshape(-1)
    ys = x[..., 1].reshape(-1)
    tab = textures.reshape(FEAT, TEX * TEX).T

    one_set = [
        pltpu.VMEM((W_OUT,), jnp.float32),
        pltpu.VMEM((W_OUT,), jnp.float32),
        pltpu.VMEM((QUARTERS, CHUNK), jnp.int32),
        pltpu.VMEM((QUARTERS, CHUNK), jnp.int32),
        pltpu.VMEM((QUARTERS, CHUNK), jnp.int32),
        pltpu.VMEM((QUARTERS, CHUNK), jnp.int32),
        pltpu.VMEM((W_OUT,), jnp.float32),
        pltpu.VMEM((W_OUT,), jnp.float32),
        pltpu.VMEM((W_OUT,), jnp.float32),
        pltpu.VMEM((W_OUT,), jnp.float32),
        pltpu.VMEM((W_OUT, FEAT), jnp.float32),
        pltpu.VMEM((W_OUT, FEAT), jnp.float32),
        pltpu.VMEM((W_OUT, FEAT), jnp.float32),
        pltpu.VMEM((W_OUT, FEAT), jnp.float32),
        pltpu.VMEM((FEAT, OPITCH), jnp.float32),
    ]

    mesh = plsc.VectorSubcoreMesh(core_axis_name="c", subcore_axis_name="s")
    f = pl.kernel(
        functools.partial(_body, rows_per_w=rows_per_w),
        out_type=jax.ShapeDtypeStruct((batch, FEAT, TEX, TEX), jnp.float32),
        mesh=mesh,
        compiler_params=pltpu.CompilerParams(
            needs_layout_passes=False, use_tc_tiling_on_sc=False),
        scratch_types=one_set + one_set + [
            pltpu.SemaphoreType.DMA((2, QUARTERS)),
            pltpu.SemaphoreType.DMA((2,)),
            pltpu.SemaphoreType.DMA((2,)),
        ],
    )
    return f(xs, ys, tab)


# speedup vs baseline: 1.4939x; 1.0615x over previous
import functools

import jax
import jax.numpy as jnp
from jax import lax
from jax.experimental import pallas as pl
from jax.experimental.pallas import tpu as pltpu
from jax.experimental.pallas import tpu_sc as plsc

FEAT = 16
TEX = 512
L = 16
NW = 32
CHUNK = 128
W_OUT = 512
QUARTERS = W_OUT // CHUNK
OPITCH = W_OUT + 1


def _bcast(vec, p):
    idx = jnp.full((L, 1), p, jnp.int32)
    return lax.gather(
        vec, idx,
        lax.GatherDimensionNumbers(
            offset_dims=(), collapsed_slice_dims=(0,), start_index_map=(0,)),
        (1,), mode=lax.GatherScatterMode.PROMISE_IN_BOUNDS)


def _body(xs_hbm, ys_hbm, tab_hbm, out_hbm, *scr, rows_per_w):
    bufs = [scr[0:15], scr[15:30]]
    gsem, csem, osem = scr[30], scr[31], scr[32]
    cid = lax.axis_index("c")
    sid = lax.axis_index("s")
    wid = sid * 2 + cid
    row0 = wid * rows_per_w

    def coords_fire(rglob, par):
        xsv, ysv = bufs[par][0], bufs[par][1]
        base = rglob * W_OUT
        pltpu.async_copy(xs_hbm.at[pl.ds(base, W_OUT)], xsv, csem.at[par])
        pltpu.async_copy(ys_hbm.at[pl.ds(base, W_OUT)], ysv, csem.at[par])

    def coords_wait(rglob, par):
        xsv, ysv = bufs[par][0], bufs[par][1]
        base = rglob * W_OUT
        pltpu.make_async_copy(
            xs_hbm.at[pl.ds(base, W_OUT)], xsv, csem.at[par]).wait()
        pltpu.make_async_copy(
            ys_hbm.at[pl.ds(base, W_OUT)], ysv, csem.at[par]).wait()

    def do_row(r_local, par):
        xsv, ysv = bufs[par][0], bufs[par][1]
        ibs = bufs[par][2:6]
        wbs = bufs[par][6:10]
        rbs = bufs[par][10:14]
        ob = bufs[par][14]
        i00, i01, i10, i11 = ibs
        w00, w01, w10, w11 = wbs
        r00, r01, r10, r11 = rbs

        rglob = row0 + r_local
        n = rglob // TEX
        h = rglob % TEX

        coords_wait(rglob, par)

        @pl.when(r_local + 1 < rows_per_w)
        def _():
            coords_fire(row0 + r_local + 1, 1 - par)

        for q in range(QUARTERS):
            def stage1(g, _):
                gsl = pl.ds(q * CHUNK + g * L, L)
                xv = xsv[gsl]
                yv = ysv[gsl]
                gx = xv * 2.0 - 1.0
                gy = yv * 2.0 - 1.0
                ix = ((gx + 1.0) * TEX - 1.0) * 0.5
                iy = ((gy + 1.0) * TEX - 1.0) * 0.5
                ix0 = (ix + 1.0).astype(jnp.int32) - 1
                iy0 = (iy + 1.0).astype(jnp.int32) - 1
                fx = ix - ix0.astype(jnp.float32)
                fy = iy - iy0.astype(jnp.float32)
                ix1 = ix0 + 1
                iy1 = iy0 + 1
                zero = jnp.zeros((L,), jnp.float32)
                wx0 = jnp.where(ix0 >= 0, 1.0 - fx, zero)
                wx1 = jnp.where(ix1 <= TEX - 1, fx, zero)
                wy0 = jnp.where(iy0 >= 0, 1.0 - fy, zero)
                wy1 = jnp.where(iy1 <= TEX - 1, fy, zero)
                cx0 = jnp.maximum(ix0, 0)
                cx1 = jnp.minimum(ix1, TEX - 1)
                ry0 = jnp.maximum(iy0, 0) * TEX
                ry1 = jnp.minimum(iy1, TEX - 1) * TEX
                sl = pl.ds(g * L, L)
                i00[q, sl] = ry0 + cx0
                i01[q, sl] = ry0 + cx1
                i10[q, sl] = ry1 + cx0
                i11[q, sl] = ry1 + cx1
                w00[gsl] = wy0 * wx0
                w01[gsl] = wy0 * wx1
                w10[gsl] = wy1 * wx0
                w11[gsl] = wy1 * wx1
                return 0

            lax.fori_loop(0, CHUNK // L, stage1, 0)

            rsl = pl.ds(q * CHUNK, CHUNK)
            pltpu.async_copy(tab_hbm.at[i00.at[q]], r00.at[rsl], gsem.at[par, q])
            pltpu.async_copy(tab_hbm.at[i01.at[q]], r01.at[rsl], gsem.at[par, q])
            pltpu.async_copy(tab_hbm.at[i10.at[q]], r10.at[rsl], gsem.at[par, q])
            pltpu.async_copy(tab_hbm.at[i11.at[q]], r11.at[rsl], gsem.at[par, q])

        @pl.when(r_local >= 2)
        def _():
            pltpu.make_async_copy(
                ob.at[:, pl.ds(0, W_OUT)], out_hbm.at[n, :, h, :],
                osem.at[par]).wait()

        for q in range(QUARTERS):
            rsl = pl.ds(q * CHUNK, CHUNK)
            for rbuf, ibuf in ((r00, i00), (r01, i01), (r10, i10), (r11, i11)):
                pltpu.make_async_copy(
                    tab_hbm.at[ibuf.at[q]], rbuf.at[rsl],
                    gsem.at[par, q]).wait()

            @plsc.parallel_loop(0, CHUNK // L, 1, unroll=1)
            def stage2(g):
                sl = pl.ds(q * CHUNK + g * L, L)
                a00 = w00[sl]
                a01 = w01[sl]
                a10 = w10[sl]
                a11 = w11[sl]
                lanes = lax.iota(jnp.int32, L)
                col0 = jnp.full((L,), q * CHUNK + g * L, jnp.int32)
                for p in range(L):
                    b00 = _bcast(a00, p)
                    b01 = _bcast(a01, p)
                    b10 = _bcast(a10, p)
                    b11 = _bcast(a11, p)
                    pt = q * CHUNK + g * L + p
                    v00 = r00[pt]
                    v01 = r01[pt]
                    v10 = r10[pt]
                    v11 = r11[pt]
                    acc = b00 * v00 + b01 * v01 + b10 * v10 + b11 * v11
                    plsc.store_scatter(ob, [lanes, col0 + p], acc)

        pltpu.async_copy(ob.at[:, pl.ds(0, W_OUT)], out_hbm.at[n, :, h, :],
                         osem.at[par])

    coords_fire(row0, 0)

    def row_pair(i, _):
        do_row(2 * i, 0)
        do_row(2 * i + 1, 1)
        return 0

    lax.fori_loop(0, rows_per_w // 2, row_pair, 0)

    for par in (0, 1):
        r_local = rows_per_w - 2 + par
        rglob = row0 + r_local
        n = rglob // TEX
        h = rglob % TEX
        ob = bufs[par][14]
        pltpu.make_async_copy(
            ob.at[:, pl.ds(0, W_OUT)], out_hbm.at[n, :, h, :],
            osem.at[par]).wait()


def kernel(x, textures):
    batch = x.shape[0]
    rows = batch * TEX
    rows_per_w = rows // NW

    xs = x[..., 0].reshape(-1)
    ys = x[..., 1].reshape(-1)
    tab = textures.reshape(FEAT, TEX * TEX).T

    one_set = [
        pltpu.VMEM((W_OUT,), jnp.float32),
        pltpu.VMEM((W_OUT,), jnp.float32),
        pltpu.VMEM((QUARTERS, CHUNK), jnp.int32),
        pltpu.VMEM((QUARTERS, CHUNK), jnp.int32),
        pltpu.VMEM((QUARTERS, CHUNK), jnp.int32),
        pltpu.VMEM((QUARTERS, CHUNK), jnp.int32),
        pltpu.VMEM((W_OUT,), jnp.float32),
        pltpu.VMEM((W_OUT,), jnp.float32),
        pltpu.VMEM((W_OUT,), jnp.float32),
        pltpu.VMEM((W_OUT,), jnp.float32),
        pltpu.VMEM((W_OUT, FEAT), jnp.float32),
        pltpu.VMEM((W_OUT, FEAT), jnp.float32),
        pltpu.VMEM((W_OUT, FEAT), jnp.float32),
        pltpu.VMEM((W_OUT, FEAT), jnp.float32),
        pltpu.VMEM((FEAT, OPITCH), jnp.float32),
    ]

    mesh = plsc.VectorSubcoreMesh(core_axis_name="c", subcore_axis_name="s")
    f = pl.kernel(
        functools.partial(_body, rows_per_w=rows_per_w),
        out_type=jax.ShapeDtypeStruct((batch, FEAT, TEX, TEX), jnp.float32),
        mesh=mesh,
        compiler_params=pltpu.CompilerParams(
            needs_layout_passes=False, use_tc_tiling_on_sc=False),
        scratch_types=one_set + one_set + [
            pltpu.SemaphoreType.DMA((2, QUARTERS)),
            pltpu.SemaphoreType.DMA((2,)),
            pltpu.SemaphoreType.DMA((2,)),
        ],
    )
    return f(xs, ys, tab)
